# Initial kernel scaffold; baseline (speedup 1.0000x reference)
import jax
import jax.numpy as jnp
from jax.experimental import pallas as pl


def kernel(in_embeddings, adj_lists, batch_indices):
    raise NotImplementedError("write your pallas kernel here")


# speedup vs baseline: 1.4273x; 1.4273x over previous
import functools

import jax
import jax.numpy as jnp
from jax import lax
from jax.experimental import pallas as pl
from jax.experimental.pallas import tpu as pltpu
from jax.experimental.pallas import tpu_sc as plsc

_NC = 2
_NS = 16
_NW = _NC * _NS
_L = 16

_CR = 8
_CI = _CR * 16


def _worker_id():
    return lax.axis_index("s") * _NC + lax.axis_index("c")


@functools.lru_cache(maxsize=None)
def _build_agg_kernel(NP, DEG, D):
    BPW = NP // _NW
    NCH = BPW // _CR
    assert NCH % 2 == 0 and DEG == 16 and D % _L == 0
    inv_deg = 1.0 / DEG

    mesh = plsc.VectorSubcoreMesh(core_axis_name="c", subcore_axis_name="s")

    @functools.partial(
        pl.kernel,
        out_type=jax.ShapeDtypeStruct((NP, D), jnp.float32),
        mesh=mesh,
        scratch_types=[
            pltpu.VMEM((BPW * DEG,), jnp.int32),
            pltpu.VMEM((2, _CI, D), jnp.float32),
            pltpu.VMEM((2, _CR, D), jnp.float32),
            pltpu.SemaphoreType.DMA,
            pltpu.SemaphoreType.DMA,
            pltpu.SemaphoreType.DMA,
            pltpu.SemaphoreType.DMA,
        ],
    )
    def agg_kernel(adj_hbm, emb_hbm, agg_hbm, idx_v, rows_v, acc_v,
                   gsem0, gsem1, osem0, osem1):
        base = _worker_id() * BPW
        pltpu.sync_copy(adj_hbm.at[pl.ds(base * DEG, BPW * DEG)], idx_v)

        def start_gather(c, b, sem):
            pltpu.async_copy(
                emb_hbm.at[idx_v.at[pl.ds(c * _CI, _CI)]], rows_v.at[b], sem)

        start_gather(0, 0, gsem0)
        start_gather(1, 1, gsem1)

        def pair(p, carry):
            for b, gsem, osem in ((0, gsem0, osem0), (1, gsem1, osem1)):
                c = p * 2 + b
                pltpu.make_async_copy(
                    emb_hbm.at[idx_v.at[pl.ds(0, _CI)]], rows_v.at[b],
                    gsem).wait()

                @pl.when(c >= 2)
                def _():
                    pltpu.make_async_copy(
                        acc_v.at[b], agg_hbm.at[pl.ds(0, _CR)], osem).wait()

                rows = rows_v.at[b]
                acc = acc_v.at[b]

                def row_body(r, carry2):
                    rb = r * DEG
                    for l in range(D // _L):
                        s = pl.ds(l * _L, _L)
                        v = rows[rb, s]
                        for j in range(1, DEG):
                            v = v + rows[rb + j, s]
                        acc[r, s] = v * inv_deg
                    return carry2

                lax.fori_loop(0, _CR, row_body, 0)

                pltpu.async_copy(
                    acc_v.at[b], agg_hbm.at[pl.ds(base + c * _CR, _CR)], osem)

                @pl.when(c + 2 < NCH)
                def _():
                    start_gather(c + 2, b, gsem)
            return carry

        lax.fori_loop(0, NCH // 2, pair, 0)

        pltpu.make_async_copy(acc_v.at[0], agg_hbm.at[pl.ds(0, _CR)],
                              osem0).wait()
        pltpu.make_async_copy(acc_v.at[1], agg_hbm.at[pl.ds(0, _CR)],
                              osem1).wait()

    return agg_kernel


@functools.lru_cache(maxsize=None)
def _build_row_gather_kernel(NP, D):
    BPW = NP // _NW
    GC = 64
    assert BPW % GC == 0

    mesh = plsc.VectorSubcoreMesh(core_axis_name="c", subcore_axis_name="s")

    @functools.partial(
        pl.kernel,
        out_type=jax.ShapeDtypeStruct((NP, D), jnp.float32),
        mesh=mesh,
        scratch_types=[
            pltpu.VMEM((BPW,), jnp.int32),
            pltpu.VMEM((BPW, D), jnp.float32),
            pltpu.SemaphoreType.DMA,
        ],
    )
    def gather_kernel(idx_hbm, table_hbm, out_hbm, idx_v, rows_v, sem):
        base = _worker_id() * BPW
        pltpu.sync_copy(idx_hbm.at[pl.ds(base, BPW)], idx_v)
        cps = []
        for k in range(BPW // GC):
            cps.append(pltpu.async_copy(
                table_hbm.at[idx_v.at[pl.ds(k * GC, GC)]],
                rows_v.at[pl.ds(k * GC, GC)], sem))
        for cp in cps:
            cp.wait()
        pltpu.sync_copy(rows_v, out_hbm.at[pl.ds(base, BPW)])

    return gather_kernel


def kernel(in_embeddings, adj_lists, batch_indices):
    N, D = in_embeddings.shape
    B = batch_indices.shape[0]
    DEG = adj_lists.shape[1]

    align = _NW * _CR * 2
    NP = ((max(N, B) + align - 1) // align) * align

    adj = adj_lists.astype(jnp.int32)
    bidx = batch_indices.astype(jnp.int32)
    adj_flat = jnp.pad(adj, ((0, NP - N), (0, 0))).reshape(NP * DEG)
    bidx_pad = jnp.pad(bidx, (0, NP - B))

    agg = _build_agg_kernel(NP, DEG, D)(adj_flat, in_embeddings)
    out = _build_row_gather_kernel(NP, D)(bidx_pad, agg)
    return out[:B]


# speedup vs baseline: 1.4426x; 1.0107x over previous
import functools

import jax
import jax.numpy as jnp
from jax import lax
from jax.experimental import pallas as pl
from jax.experimental.pallas import tpu as pltpu
from jax.experimental.pallas import tpu_sc as plsc

_NC = 2
_NS = 16
_NW = _NC * _NS
_L = 16

_CR = 8
_CI = _CR * 16


def _worker_id():
    return lax.axis_index("s") * _NC + lax.axis_index("c")


@functools.lru_cache(maxsize=None)
def _build_agg_kernel(NP, DEG, D):
    BPW = NP // _NW
    NCH = BPW // _CR
    assert NCH % 2 == 0 and DEG == 16 and D % _L == 0
    inv_deg = 1.0 / DEG

    mesh = plsc.VectorSubcoreMesh(core_axis_name="c", subcore_axis_name="s")

    @functools.partial(
        pl.kernel,
        out_type=jax.ShapeDtypeStruct((NP, D), jnp.float32),
        mesh=mesh,
        scratch_types=[
            pltpu.VMEM((BPW * DEG,), jnp.int32),
            pltpu.VMEM((2, _CI, D), jnp.float32),
            pltpu.VMEM((2, _CR, D), jnp.float32),
            pltpu.SemaphoreType.DMA,
            pltpu.SemaphoreType.DMA,
            pltpu.SemaphoreType.DMA,
            pltpu.SemaphoreType.DMA,
        ],
    )
    def agg_kernel(adj_hbm, emb_hbm, agg_hbm, idx_v, rows_v, acc_v,
                   gsem0, gsem1, osem0, osem1):
        base = _worker_id() * BPW
        pltpu.sync_copy(adj_hbm.at[pl.ds(base * DEG, BPW * DEG)], idx_v)

        def start_gather(c, b, sem):
            pltpu.async_copy(
                emb_hbm.at[idx_v.at[pl.ds(c * _CI, _CI)]], rows_v.at[b], sem)

        start_gather(0, 0, gsem0)
        start_gather(1, 1, gsem1)

        def pair(p, carry):
            for b, gsem, osem in ((0, gsem0, osem0), (1, gsem1, osem1)):
                c = p * 2 + b
                pltpu.make_async_copy(
                    emb_hbm.at[idx_v.at[pl.ds(0, _CI)]], rows_v.at[b],
                    gsem).wait()

                @pl.when(c >= 2)
                def _():
                    pltpu.make_async_copy(
                        acc_v.at[b], agg_hbm.at[pl.ds(0, _CR)], osem).wait()

                rows = rows_v.at[b]
                acc = acc_v.at[b]

                def row_body(r, carry2):
                    rb = r * DEG
                    for l in range(D // _L):
                        s = pl.ds(l * _L, _L)
                        vals = [rows[rb + j, s] for j in range(DEG)]
                        while len(vals) > 1:
                            vals = [vals[i] + vals[i + 1]
                                    for i in range(0, len(vals), 2)]
                        acc[r, s] = vals[0] * inv_deg
                    return carry2

                lax.fori_loop(0, _CR, row_body, 0)

                pltpu.async_copy(
                    acc_v.at[b], agg_hbm.at[pl.ds(base + c * _CR, _CR)], osem)

                @pl.when(c + 2 < NCH)
                def _():
                    start_gather(c + 2, b, gsem)
            return carry

        lax.fori_loop(0, NCH // 2, pair, 0)

        pltpu.make_async_copy(acc_v.at[0], agg_hbm.at[pl.ds(0, _CR)],
                              osem0).wait()
        pltpu.make_async_copy(acc_v.at[1], agg_hbm.at[pl.ds(0, _CR)],
                              osem1).wait()

    return agg_kernel


@functools.lru_cache(maxsize=None)
def _build_row_gather_kernel(NP, D):
    BPW = NP // _NW
    GC = 64
    assert BPW % GC == 0

    mesh = plsc.VectorSubcoreMesh(core_axis_name="c", subcore_axis_name="s")

    @functools.partial(
        pl.kernel,
        out_type=jax.ShapeDtypeStruct((NP, D), jnp.float32),
        mesh=mesh,
        scratch_types=[
            pltpu.VMEM((BPW,), jnp.int32),
            pltpu.VMEM((BPW, D), jnp.float32),
            pltpu.SemaphoreType.DMA,
        ],
    )
    def gather_kernel(idx_hbm, table_hbm, out_hbm, idx_v, rows_v, sem):
        base = _worker_id() * BPW
        pltpu.sync_copy(idx_hbm.at[pl.ds(base, BPW)], idx_v)
        cps = []
        for k in range(BPW // GC):
            cps.append(pltpu.async_copy(
                table_hbm.at[idx_v.at[pl.ds(k * GC, GC)]],
                rows_v.at[pl.ds(k * GC, GC)], sem))
        for cp in cps:
            cp.wait()
        pltpu.sync_copy(rows_v, out_hbm.at[pl.ds(base, BPW)])

    return gather_kernel


def kernel(in_embeddings, adj_lists, batch_indices):
    N, D = in_embeddings.shape
    B = batch_indices.shape[0]
    DEG = adj_lists.shape[1]

    align = _NW * _CR * 2
    NP = ((max(N, B) + align - 1) // align) * align

    adj = adj_lists.astype(jnp.int32)
    bidx = batch_indices.astype(jnp.int32)
    adj_flat = jnp.pad(adj, ((0, NP - N), (0, 0))).reshape(NP * DEG)
    bidx_pad = jnp.pad(bidx, (0, NP - B))

    agg = _build_agg_kernel(NP, DEG, D)(adj_flat, in_embeddings)
    out = _build_row_gather_kernel(NP, D)(bidx_pad, agg)
    return out[:B]
